# baseline (speedup 1.0000x reference)
import jax
import jax.numpy as jnp
from jax.experimental import pallas as pl
from jax.experimental.pallas import tpu as pltpu
from jax.experimental.pallas import tpu_sc as plsc

_VOCAB = 100000
_EMBED = 64
_HIDDEN = 512
_NPREV = 20
_BATCH = 1024

_GW = 128
_BN = 2048
_PAIR = 2 * _EMBED


def _sc_gather_pairs(table_pairs, pair_idx):
    n = pair_idx.shape[0]
    idx2 = pair_idx.reshape(1, n)
    mesh = plsc.VectorSubcoreMesh(core_axis_name="c", subcore_axis_name="s")

    @pl.kernel(out_type=jax.ShapeDtypeStruct((n, _PAIR), table_pairs.dtype),
               mesh=mesh)
    def gather_kernel(table_hbm, idx_hbm, out_hbm):
        def body(i_vmem, o_vmem):
            pltpu.sync_copy(table_hbm.at[i_vmem.at[0]], o_vmem)

        pltpu.emit_pipeline(
            body,
            grid=(n // _GW,),
            in_specs=[pl.BlockSpec((1, _GW), lambda i: (0, i))],
            out_specs=[pl.BlockSpec((_GW, _PAIR), lambda i: (i, 0))],
            core_axis_name=("c", "s"),
            dimension_semantics=(pltpu.PARALLEL,),
        )(idx_hbm, out_hbm)

    return gather_kernel(table_pairs, idx2)


def _mlp_body(g_ref, idx_ref, w1p_ref, b1_ref, w2_ref, b2_ref, out_ref,
              h_ref):
    j = pl.program_id(0)

    @pl.when(j == 0)
    def _():
        half = jax.lax.broadcasted_iota(
            jnp.int32, (_BATCH, _PAIR), 1) // _EMBED
        acc = jnp.zeros((_BATCH, _HIDDEN), jnp.float32)
        for t in range(_NPREV):
            gt = g_ref[:, t * _PAIR:(t + 1) * _PAIR]
            pt = idx_ref[:, t:t + 1] & 1
            sel = jnp.where(half == pt, gt, 0.0)
            acc = acc + jnp.dot(sel, w1p_ref[t],
                                preferred_element_type=jnp.float32)
        h_ref[...] = jnp.tanh(acc + b1_ref[...]).astype(jnp.bfloat16)

    out_ref[...] = (
        jnp.dot(h_ref[...], w2_ref[...].astype(jnp.bfloat16),
                preferred_element_type=jnp.float32)
        + b2_ref[...])


def _mlp(g, idx, W1p, b1, W2, b2):
    nblk = pl.cdiv(_VOCAB, _BN)
    return pl.pallas_call(
        _mlp_body,
        grid=(nblk,),
        in_specs=[
            pl.BlockSpec((_BATCH, _NPREV * _PAIR), lambda j: (0, 0)),
            pl.BlockSpec((_BATCH, _NPREV), lambda j: (0, 0)),
            pl.BlockSpec((_NPREV, _PAIR, _HIDDEN), lambda j: (0, 0, 0)),
            pl.BlockSpec((_HIDDEN,), lambda j: (0,)),
            pl.BlockSpec((_HIDDEN, _BN), lambda j: (0, j)),
            pl.BlockSpec((_BN,), lambda j: (j,)),
        ],
        out_specs=pl.BlockSpec((_BATCH, _BN), lambda j: (0, j)),
        out_shape=jax.ShapeDtypeStruct((_BATCH, _VOCAB), jnp.float32),
        scratch_shapes=[pltpu.VMEM((_BATCH, _HIDDEN), jnp.bfloat16)],
    )(g, idx, W1p, b1, W2, b2)


def kernel(inputs, embed_table, W1, b1, W2, b2):
    flat_idx = inputs.reshape(-1)
    table_pairs = embed_table.reshape(_VOCAB // 2, _PAIR)
    gathered = _sc_gather_pairs(table_pairs, flat_idx >> 1)
    g = gathered.reshape(_BATCH, _NPREV * _PAIR)
    W1r = W1.reshape(_NPREV, _EMBED, _HIDDEN)
    W1p = jnp.concatenate([W1r, W1r], axis=1)
    return _mlp(g, inputs, W1p, b1, W2, b2)


# speedup vs baseline: 1.0014x; 1.0014x over previous
import jax
import jax.numpy as jnp
from jax.experimental import pallas as pl
from jax.experimental.pallas import tpu as pltpu
from jax.experimental.pallas import tpu_sc as plsc

_VOCAB = 100000
_EMBED = 64
_HIDDEN = 512
_NPREV = 20
_BATCH = 1024

_GW = 128
_BN = 2048
_PAIR = 2 * _EMBED


def _sc_gather_pairs(table_pairs, pair_idx):
    n = pair_idx.shape[0]
    idx2 = pair_idx.reshape(1, n)
    mesh = plsc.VectorSubcoreMesh(core_axis_name="c", subcore_axis_name="s")

    @pl.kernel(out_type=jax.ShapeDtypeStruct((n, _PAIR), table_pairs.dtype),
               mesh=mesh)
    def gather_kernel(table_hbm, idx_hbm, out_hbm):
        def body(i_vmem, o_vmem):
            pltpu.sync_copy(table_hbm.at[i_vmem.at[0]], o_vmem)

        pltpu.emit_pipeline(
            body,
            grid=(n // _GW,),
            in_specs=[pl.BlockSpec((1, _GW), lambda i: (0, i))],
            out_specs=[pl.BlockSpec((_GW, _PAIR), lambda i: (i, 0))],
            core_axis_name=("c", "s"),
            dimension_semantics=(pltpu.PARALLEL,),
        )(idx_hbm, out_hbm)

    return gather_kernel(table_pairs, idx2)


def _h_body(g_ref, idx_ref, w1p_ref, b1_ref, h_ref):
    half = jax.lax.broadcasted_iota(
        jnp.int32, (_BATCH, _PAIR), 1) // _EMBED
    acc = jnp.zeros((_BATCH, _HIDDEN), jnp.float32)
    for t in range(_NPREV):
        gt = g_ref[:, t * _PAIR:(t + 1) * _PAIR]
        pt = idx_ref[:, t:t + 1] & 1
        sel = jnp.where(half == pt, gt, 0.0)
        acc = acc + jnp.dot(sel, w1p_ref[t],
                            preferred_element_type=jnp.float32)
    h_ref[...] = jnp.tanh(acc + b1_ref[...]).astype(jnp.bfloat16)


def _h_layer(g, idx, W1p, b1):
    return pl.pallas_call(
        _h_body,
        out_shape=jax.ShapeDtypeStruct((_BATCH, _HIDDEN), jnp.bfloat16),
    )(g, idx, W1p, b1)


def _out_body(h_ref, w2_ref, b2_ref, out_ref):
    out_ref[...] = (
        jnp.dot(h_ref[...], w2_ref[...].astype(jnp.bfloat16),
                preferred_element_type=jnp.float32)
        + b2_ref[...])


def _out_layer(h, W2, b2):
    nblk = pl.cdiv(_VOCAB, _BN)
    return pl.pallas_call(
        _out_body,
        grid=(nblk,),
        in_specs=[
            pl.BlockSpec((_BATCH, _HIDDEN), lambda j: (0, 0)),
            pl.BlockSpec((_HIDDEN, _BN), lambda j: (0, j)),
            pl.BlockSpec((_BN,), lambda j: (j,)),
        ],
        out_specs=pl.BlockSpec((_BATCH, _BN), lambda j: (0, j)),
        out_shape=jax.ShapeDtypeStruct((_BATCH, _VOCAB), jnp.float32),
        compiler_params=pltpu.CompilerParams(
            dimension_semantics=("parallel",)),
    )(h, W2, b2)


def kernel(inputs, embed_table, W1, b1, W2, b2):
    flat_idx = inputs.reshape(-1)
    table_pairs = embed_table.reshape(_VOCAB // 2, _PAIR)
    gathered = _sc_gather_pairs(table_pairs, flat_idx >> 1)
    g = gathered.reshape(_BATCH, _NPREV * _PAIR)
    W1r = W1.reshape(_NPREV, _EMBED, _HIDDEN)
    W1p = jnp.concatenate([W1r, W1r], axis=1)
    h = _h_layer(g, inputs, W1p, b1)
    return _out_layer(h, W2, b2)


# speedup vs baseline: 2.2927x; 2.2895x over previous
import jax
import jax.numpy as jnp
from jax.experimental import pallas as pl
from jax.experimental.pallas import tpu as pltpu
from jax.experimental.pallas import tpu_sc as plsc

_VOCAB = 100000
_EMBED = 64
_HIDDEN = 512
_NPREV = 20
_BATCH = 1024

_GW = 128
_BN = 2048
_PAIR = 2 * _EMBED


def _sc_gather_pairs(table_pairs, pair_idx):
    n = pair_idx.shape[0]
    idx2 = pair_idx.reshape(1, n)
    mesh = plsc.VectorSubcoreMesh(core_axis_name="c", subcore_axis_name="s")

    @pl.kernel(out_type=jax.ShapeDtypeStruct((n, _PAIR), table_pairs.dtype),
               mesh=mesh)
    def gather_kernel(table_hbm, idx_hbm, out_hbm):
        def body(i_vmem, o_vmem):
            pltpu.sync_copy(table_hbm.at[i_vmem.at[0]], o_vmem)

        pltpu.emit_pipeline(
            body,
            grid=(n // _GW,),
            in_specs=[pl.BlockSpec((1, _GW), lambda i: (0, i))],
            out_specs=[pl.BlockSpec((_GW, _PAIR), lambda i: (i, 0))],
            core_axis_name=("c", "s"),
            dimension_semantics=(pltpu.PARALLEL,),
        )(idx_hbm, out_hbm)

    return gather_kernel(table_pairs, idx2)


def _h_body(g_ref, idx_ref, w1p_ref, b1_ref, ht_ref):
    half = jax.lax.broadcasted_iota(
        jnp.int32, (_BATCH, _PAIR), 1) // _EMBED
    acc = jnp.zeros((_BATCH, _HIDDEN), jnp.float32)
    for t in range(_NPREV):
        gt = g_ref[:, t * _PAIR:(t + 1) * _PAIR]
        pt = idx_ref[:, t:t + 1] & 1
        sel = jnp.where(half == pt, gt, 0.0)
        acc = acc + jnp.dot(sel, w1p_ref[t],
                            preferred_element_type=jnp.float32)
    h = jnp.tanh(acc + b1_ref[...])
    ht_ref[...] = h.T.astype(jnp.bfloat16)


def _h_layer(g, idx, W1p, b1):
    return pl.pallas_call(
        _h_body,
        out_shape=jax.ShapeDtypeStruct((_HIDDEN, _BATCH), jnp.bfloat16),
    )(g, idx, W1p, b1)


def _out_body(ht_ref, w2t_ref, b2c_ref, out_ref):
    out_ref[...] = (
        jnp.dot(w2t_ref[...].astype(jnp.bfloat16), ht_ref[...],
                preferred_element_type=jnp.float32)
        + b2c_ref[...])


def _out_layer_t(ht, W2t, b2c):
    nblk = pl.cdiv(_VOCAB, _BN)
    return pl.pallas_call(
        _out_body,
        grid=(nblk,),
        in_specs=[
            pl.BlockSpec((_HIDDEN, _BATCH), lambda j: (0, 0)),
            pl.BlockSpec((_BN, _HIDDEN), lambda j: (j, 0)),
            pl.BlockSpec((_BN, 1), lambda j: (j, 0)),
        ],
        out_specs=pl.BlockSpec((_BN, _BATCH), lambda j: (j, 0)),
        out_shape=jax.ShapeDtypeStruct((_VOCAB, _BATCH), jnp.float32),
        compiler_params=pltpu.CompilerParams(
            dimension_semantics=("arbitrary",)),
    )(ht, W2t, b2c)


def kernel(inputs, embed_table, W1, b1, W2, b2):
    flat_idx = inputs.reshape(-1)
    table_pairs = embed_table.reshape(_VOCAB // 2, _PAIR)
    gathered = _sc_gather_pairs(table_pairs, flat_idx >> 1)
    g = gathered.reshape(_BATCH, _NPREV * _PAIR)
    W1r = W1.reshape(_NPREV, _EMBED, _HIDDEN)
    W1p = jnp.concatenate([W1r, W1r], axis=1)
    ht = _h_layer(g, inputs, W1p, b1)
    out_t = _out_layer_t(ht, W2.T, b2[:, None])
    return out_t.T


# speedup vs baseline: 2.7607x; 1.2041x over previous
import jax
import jax.numpy as jnp
from jax.experimental import pallas as pl
from jax.experimental.pallas import tpu as pltpu
from jax.experimental.pallas import tpu_sc as plsc

_VOCAB = 100000
_EMBED = 64
_HIDDEN = 512
_NPREV = 20
_BATCH = 1024

_GW = 128
_BN = 2048
_PAIR = 2 * _EMBED


def _sc_gather_pairs(table_pairs, pair_idx):
    n = pair_idx.shape[0]
    idx2 = pair_idx.reshape(1, n)
    mesh = plsc.VectorSubcoreMesh(core_axis_name="c", subcore_axis_name="s")

    @pl.kernel(out_type=jax.ShapeDtypeStruct((n, _PAIR), table_pairs.dtype),
               mesh=mesh)
    def gather_kernel(table_hbm, idx_hbm, out_hbm):
        def body(i_vmem, o_vmem):
            pltpu.sync_copy(table_hbm.at[i_vmem.at[0]], o_vmem)

        pltpu.emit_pipeline(
            body,
            grid=(n // _GW,),
            in_specs=[pl.BlockSpec((1, _GW), lambda i: (0, i))],
            out_specs=[pl.BlockSpec((_GW, _PAIR), lambda i: (i, 0))],
            core_axis_name=("c", "s"),
            dimension_semantics=(pltpu.PARALLEL,),
        )(idx_hbm, out_hbm)

    return gather_kernel(table_pairs, idx2)


def _h_body(g_ref, idx_ref, w1p_ref, b1_ref, ht_ref):
    half = jax.lax.broadcasted_iota(
        jnp.int32, (_BATCH, _PAIR), 1) // _EMBED
    acc = jnp.zeros((_BATCH, _HIDDEN), jnp.float32)
    for t in range(_NPREV):
        gt = g_ref[t * _BATCH:(t + 1) * _BATCH, :]
        pt = idx_ref[:, t:t + 1] & 1
        sel = jnp.where(half == pt, gt, 0.0)
        acc = acc + jnp.dot(sel, w1p_ref[t],
                            preferred_element_type=jnp.float32)
    h = jnp.tanh(acc + b1_ref[...])
    ht_ref[...] = h.T.astype(jnp.bfloat16)


def _h_layer(g_tm, idx, W1p, b1):
    return pl.pallas_call(
        _h_body,
        out_shape=jax.ShapeDtypeStruct((_HIDDEN, _BATCH), jnp.bfloat16),
    )(g_tm, idx, W1p, b1)


def _out_body(ht_ref, w2t_ref, b2_ref, out_ref):
    out_ref[...] = (
        jnp.dot(w2t_ref[...].astype(jnp.bfloat16), ht_ref[...],
                preferred_element_type=jnp.float32)
        + b2_ref[...][:, None])


def _out_layer_t(ht, W2t, b2):
    nblk = pl.cdiv(_VOCAB, _BN)
    return pl.pallas_call(
        _out_body,
        grid=(nblk,),
        in_specs=[
            pl.BlockSpec((_HIDDEN, _BATCH), lambda j: (0, 0)),
            pl.BlockSpec((_BN, _HIDDEN), lambda j: (j, 0)),
            pl.BlockSpec((_BN,), lambda j: (j,)),
        ],
        out_specs=pl.BlockSpec((_BN, _BATCH), lambda j: (j, 0)),
        out_shape=jax.ShapeDtypeStruct((_VOCAB, _BATCH), jnp.float32),
        compiler_params=pltpu.CompilerParams(
            dimension_semantics=("arbitrary",)),
    )(ht, W2t, b2)


def kernel(inputs, embed_table, W1, b1, W2, b2):
    flat_idx_tm = inputs.T.reshape(-1)
    table_pairs = embed_table.reshape(_VOCAB // 2, _PAIR)
    g_tm = _sc_gather_pairs(table_pairs, flat_idx_tm >> 1)
    W1r = W1.reshape(_NPREV, _EMBED, _HIDDEN)
    W1p = jnp.concatenate([W1r, W1r], axis=1)
    ht = _h_layer(g_tm, inputs, W1p, b1)
    out_t = _out_layer_t(ht, W2.T, b2)
    return out_t.T


# speedup vs baseline: 2.7847x; 1.0087x over previous
import jax
import jax.numpy as jnp
from jax.experimental import pallas as pl
from jax.experimental.pallas import tpu as pltpu
from jax.experimental.pallas import tpu_sc as plsc

_VOCAB = 100000
_EMBED = 64
_HIDDEN = 512
_NPREV = 20
_BATCH = 1024

_GW = 128
_BN = 3072
_PAIR = 2 * _EMBED


def _sc_gather_pairs(table_pairs, pair_idx):
    n = pair_idx.shape[0]
    idx2 = pair_idx.reshape(1, n)
    mesh = plsc.VectorSubcoreMesh(core_axis_name="c", subcore_axis_name="s")

    @pl.kernel(out_type=jax.ShapeDtypeStruct((n, _PAIR), table_pairs.dtype),
               mesh=mesh)
    def gather_kernel(table_hbm, idx_hbm, out_hbm):
        def body(i_vmem, o_vmem):
            pltpu.sync_copy(table_hbm.at[i_vmem.at[0]], o_vmem)

        pltpu.emit_pipeline(
            body,
            grid=(n // _GW,),
            in_specs=[pl.BlockSpec((1, _GW), lambda i: (0, i))],
            out_specs=[pl.BlockSpec((_GW, _PAIR), lambda i: (i, 0))],
            core_axis_name=("c", "s"),
            dimension_semantics=(pltpu.PARALLEL,),
        )(idx_hbm, out_hbm)

    return gather_kernel(table_pairs, idx2)


def _h_body(g_ref, idx_ref, w1p_ref, b1_ref, ht_ref):
    half = jax.lax.broadcasted_iota(
        jnp.int32, (_BATCH, _PAIR), 1) // _EMBED
    acc = jnp.zeros((_BATCH, _HIDDEN), jnp.float32)
    for t in range(_NPREV):
        gt = g_ref[t * _BATCH:(t + 1) * _BATCH, :]
        pt = idx_ref[:, t:t + 1] & 1
        sel = jnp.where(half == pt, gt, 0.0)
        acc = acc + jnp.dot(sel, w1p_ref[t],
                            preferred_element_type=jnp.float32)
    h = jnp.tanh(acc + b1_ref[...])
    ht_ref[...] = h.T.astype(jnp.bfloat16)


def _h_layer(g_tm, idx, W1p, b1):
    return pl.pallas_call(
        _h_body,
        out_shape=jax.ShapeDtypeStruct((_HIDDEN, _BATCH), jnp.bfloat16),
    )(g_tm, idx, W1p, b1)


def _out_body(ht_ref, w2t_ref, b2_ref, out_ref):
    out_ref[...] = (
        jnp.dot(w2t_ref[...].astype(jnp.bfloat16), ht_ref[...],
                preferred_element_type=jnp.float32)
        + b2_ref[...][:, None])


def _out_layer_t(ht, W2t, b2):
    nblk = pl.cdiv(_VOCAB, _BN)
    return pl.pallas_call(
        _out_body,
        grid=(nblk,),
        in_specs=[
            pl.BlockSpec((_HIDDEN, _BATCH), lambda j: (0, 0)),
            pl.BlockSpec((_BN, _HIDDEN), lambda j: (j, 0)),
            pl.BlockSpec((_BN,), lambda j: (j,)),
        ],
        out_specs=pl.BlockSpec((_BN, _BATCH), lambda j: (j, 0)),
        out_shape=jax.ShapeDtypeStruct((_VOCAB, _BATCH), jnp.float32),
        compiler_params=pltpu.CompilerParams(
            dimension_semantics=("arbitrary",)),
    )(ht, W2t, b2)


def kernel(inputs, embed_table, W1, b1, W2, b2):
    flat_idx_tm = inputs.T.reshape(-1)
    table_pairs = embed_table.reshape(_VOCAB // 2, _PAIR)
    g_tm = _sc_gather_pairs(table_pairs, flat_idx_tm >> 1)
    W1r = W1.reshape(_NPREV, _EMBED, _HIDDEN)
    W1p = jnp.concatenate([W1r, W1r], axis=1)
    ht = _h_layer(g_tm, inputs, W1p, b1)
    out_t = _out_layer_t(ht, W2.T, b2)
    return out_t.T


# speedup vs baseline: 2.8786x; 1.0337x over previous
import jax
import jax.numpy as jnp
from jax.experimental import pallas as pl
from jax.experimental.pallas import tpu as pltpu
from jax.experimental.pallas import tpu_sc as plsc

_VOCAB = 100000
_EMBED = 64
_HIDDEN = 512
_NPREV = 20
_BATCH = 1024

_GW = 128
_BN = 3072
_ROW = 2 * _EMBED


def _sc_gather(table_pad, flat_idx):
    n = flat_idx.shape[0]
    idx2 = flat_idx.reshape(1, n)
    mesh = plsc.VectorSubcoreMesh(core_axis_name="c", subcore_axis_name="s")

    @pl.kernel(out_type=jax.ShapeDtypeStruct((n, _ROW), table_pad.dtype),
               mesh=mesh)
    def gather_kernel(table_hbm, idx_hbm, out_hbm):
        def body(i_vmem, o_vmem):
            pltpu.sync_copy(table_hbm.at[i_vmem.at[0]], o_vmem)

        pltpu.emit_pipeline(
            body,
            grid=(n // _GW,),
            in_specs=[pl.BlockSpec((1, _GW), lambda i: (0, i))],
            out_specs=[pl.BlockSpec((_GW, _ROW), lambda i: (i, 0))],
            core_axis_name=("c", "s"),
            dimension_semantics=(pltpu.PARALLEL,),
        )(idx_hbm, out_hbm)

    return gather_kernel(table_pad, idx2)


def _h_body(g_ref, w1p_ref, b1_ref, ht_ref):
    acc = jnp.zeros((_BATCH, _HIDDEN), jnp.float32)
    for t in range(_NPREV):
        gt = g_ref[t * _BATCH:(t + 1) * _BATCH, :]
        acc = acc + jnp.dot(gt.astype(jnp.bfloat16), w1p_ref[t],
                            preferred_element_type=jnp.float32)
    h = jnp.tanh(acc + b1_ref[...])
    ht_ref[...] = h.T.astype(jnp.bfloat16)


def _h_layer(g_tm, W1p, b1):
    return pl.pallas_call(
        _h_body,
        out_shape=jax.ShapeDtypeStruct((_HIDDEN, _BATCH), jnp.bfloat16),
    )(g_tm, W1p, b1)


def _out_body(ht_ref, w2t_ref, b2_ref, out_ref):
    out_ref[...] = (
        jnp.dot(w2t_ref[...].astype(jnp.bfloat16), ht_ref[...],
                preferred_element_type=jnp.float32)
        + b2_ref[...][:, None])


def _out_layer_t(ht, W2t, b2):
    nblk = pl.cdiv(_VOCAB, _BN)
    return pl.pallas_call(
        _out_body,
        grid=(nblk,),
        in_specs=[
            pl.BlockSpec((_HIDDEN, _BATCH), lambda j: (0, 0)),
            pl.BlockSpec((_BN, _HIDDEN), lambda j: (j, 0)),
            pl.BlockSpec((_BN,), lambda j: (j,)),
        ],
        out_specs=pl.BlockSpec((_BN, _BATCH), lambda j: (j, 0)),
        out_shape=jax.ShapeDtypeStruct((_VOCAB, _BATCH), jnp.float32),
        compiler_params=pltpu.CompilerParams(
            dimension_semantics=("arbitrary",)),
    )(ht, W2t, b2)


def kernel(inputs, embed_table, W1, b1, W2, b2):
    flat_idx_tm = inputs.T.reshape(-1)
    table_pad = jnp.pad(embed_table, ((0, 0), (0, _ROW - _EMBED)))
    g_tm = _sc_gather(table_pad, flat_idx_tm)
    W1p = jnp.pad(W1.reshape(_NPREV, _EMBED, _HIDDEN),
                  ((0, 0), (0, _ROW - _EMBED), (0, 0))).astype(jnp.bfloat16)
    ht = _h_layer(g_tm, W1p, b1)
    out_t = _out_layer_t(ht, W2.T, b2)
    return out_t.T


# speedup vs baseline: 2.9391x; 1.0210x over previous
import jax
import jax.numpy as jnp
from jax.experimental import pallas as pl
from jax.experimental.pallas import tpu as pltpu
from jax.experimental.pallas import tpu_sc as plsc

_VOCAB = 100000
_EMBED = 64
_HIDDEN = 512
_NPREV = 20
_BATCH = 1024

_GW = 128
_BN = 3072
_ROW = 2 * _EMBED


def _sc_gather(table_pad, flat_idx):
    n = flat_idx.shape[0]
    idx2 = flat_idx.reshape(1, n)
    mesh = plsc.VectorSubcoreMesh(core_axis_name="c", subcore_axis_name="s")

    @pl.kernel(out_type=jax.ShapeDtypeStruct((n, _ROW), table_pad.dtype),
               mesh=mesh)
    def gather_kernel(table_hbm, idx_hbm, out_hbm):
        def body(i_vmem, o_vmem):
            pltpu.sync_copy(table_hbm.at[i_vmem.at[0]], o_vmem)

        pltpu.emit_pipeline(
            body,
            grid=(n // _GW,),
            in_specs=[pl.BlockSpec((1, _GW), lambda i: (0, i))],
            out_specs=[pl.BlockSpec((_GW, _ROW), lambda i: (i, 0))],
            core_axis_name=("c", "s"),
            dimension_semantics=(pltpu.PARALLEL,),
        )(idx_hbm, out_hbm)

    return gather_kernel(table_pad, idx2)


_CB = 2048


def _padt_body(tt_ref, out_ref):
    out_ref[:, 0:_EMBED] = tt_ref[...].T
    out_ref[:, _EMBED:] = jnp.zeros((_CB, _ROW - _EMBED), jnp.float32)


def _padt(tT):
    return pl.pallas_call(
        _padt_body,
        grid=(pl.cdiv(_VOCAB, _CB),),
        in_specs=[pl.BlockSpec((_EMBED, _CB), lambda j: (0, j))],
        out_specs=pl.BlockSpec((_CB, _ROW), lambda j: (j, 0)),
        out_shape=jax.ShapeDtypeStruct((_VOCAB, _ROW), jnp.float32),
        compiler_params=pltpu.CompilerParams(
            dimension_semantics=("arbitrary",)),
    )(tT)


def _h_body(g_ref, w1p_ref, b1_ref, ht_ref):
    acc = jnp.zeros((_BATCH, _HIDDEN), jnp.float32)
    for t in range(_NPREV):
        gt = g_ref[t * _BATCH:(t + 1) * _BATCH, :]
        acc = acc + jnp.dot(gt.astype(jnp.bfloat16), w1p_ref[t],
                            preferred_element_type=jnp.float32)
    h = jnp.tanh(acc + b1_ref[...])
    ht_ref[...] = h.T.astype(jnp.bfloat16)


def _h_layer(g_tm, W1p, b1):
    return pl.pallas_call(
        _h_body,
        out_shape=jax.ShapeDtypeStruct((_HIDDEN, _BATCH), jnp.bfloat16),
    )(g_tm, W1p, b1)


def _out_body(ht_ref, w2t_ref, b2_ref, out_ref):
    out_ref[...] = (
        jnp.dot(w2t_ref[...].astype(jnp.bfloat16), ht_ref[...],
                preferred_element_type=jnp.float32)
        + b2_ref[...][:, None])


def _out_layer_t(ht, W2t, b2):
    nblk = pl.cdiv(_VOCAB, _BN)
    return pl.pallas_call(
        _out_body,
        grid=(nblk,),
        in_specs=[
            pl.BlockSpec((_HIDDEN, _BATCH), lambda j: (0, 0)),
            pl.BlockSpec((_BN, _HIDDEN), lambda j: (j, 0)),
            pl.BlockSpec((_BN,), lambda j: (j,)),
        ],
        out_specs=pl.BlockSpec((_BN, _BATCH), lambda j: (j, 0)),
        out_shape=jax.ShapeDtypeStruct((_VOCAB, _BATCH), jnp.float32),
        compiler_params=pltpu.CompilerParams(
            dimension_semantics=("arbitrary",)),
    )(ht, W2t, b2)


def kernel(inputs, embed_table, W1, b1, W2, b2):
    flat_idx_tm = inputs.T.reshape(-1)
    table_pad = _padt(embed_table.T)
    g_tm = _sc_gather(table_pad, flat_idx_tm)
    W1p = jnp.pad(W1.reshape(_NPREV, _EMBED, _HIDDEN),
                  ((0, 0), (0, _ROW - _EMBED), (0, 0))).astype(jnp.bfloat16)
    ht = _h_layer(g_tm, W1p, b1)
    out_t = _out_layer_t(ht, W2.T, b2)
    return out_t.T


# speedup vs baseline: 3.1523x; 1.0726x over previous
import jax
import jax.numpy as jnp
from jax.experimental import pallas as pl
from jax.experimental.pallas import tpu as pltpu
from jax.experimental.pallas import tpu_sc as plsc

_VOCAB = 100000
_EMBED = 64
_HIDDEN = 512
_NPREV = 20
_BATCH = 1024

_GW = 128
_BN = 3072
_ROW = 2 * _EMBED


def _sc_gather(table_pad, flat_idx):
    n = flat_idx.shape[0]
    idx2 = flat_idx.reshape(1, n)
    mesh = plsc.VectorSubcoreMesh(core_axis_name="c", subcore_axis_name="s")

    @pl.kernel(out_type=jax.ShapeDtypeStruct((n, _ROW), table_pad.dtype),
               mesh=mesh)
    def gather_kernel(table_hbm, idx_hbm, out_hbm):
        def body(i_vmem, o_vmem):
            pltpu.sync_copy(table_hbm.at[i_vmem.at[0]], o_vmem)

        pltpu.emit_pipeline(
            body,
            grid=(n // _GW,),
            in_specs=[pl.BlockSpec((1, _GW), lambda i: (0, i))],
            out_specs=[pl.BlockSpec((_GW, _ROW), lambda i: (i, 0))],
            core_axis_name=("c", "s"),
            dimension_semantics=(pltpu.PARALLEL,),
        )(idx_hbm, out_hbm)

    return gather_kernel(table_pad, idx2)


_CB = 8192


def _padt_body(tt_ref, out_ref):
    out_ref[:, 0:_EMBED] = tt_ref[...].T
    out_ref[:, _EMBED:] = jnp.zeros((_CB, _ROW - _EMBED), jnp.float32)


def _padt(tT):
    return pl.pallas_call(
        _padt_body,
        grid=(pl.cdiv(_VOCAB, _CB),),
        in_specs=[pl.BlockSpec((_EMBED, _CB), lambda j: (0, j))],
        out_specs=pl.BlockSpec((_CB, _ROW), lambda j: (j, 0)),
        out_shape=jax.ShapeDtypeStruct((_VOCAB, _ROW), jnp.float32),
        compiler_params=pltpu.CompilerParams(
            dimension_semantics=("arbitrary",)),
    )(tT)


def _h_body(g_ref, w1p_ref, b1_ref, ht_ref):
    acc = jnp.zeros((_BATCH, _HIDDEN), jnp.float32)
    for t in range(_NPREV):
        gt = g_ref[t * _BATCH:(t + 1) * _BATCH, :]
        acc = acc + jnp.dot(gt.astype(jnp.bfloat16), w1p_ref[t],
                            preferred_element_type=jnp.float32)
    h = jnp.tanh(acc + b1_ref[...])
    ht_ref[...] = h.T.astype(jnp.bfloat16)


def _h_layer(g_tm, W1p, b1):
    return pl.pallas_call(
        _h_body,
        out_shape=jax.ShapeDtypeStruct((_HIDDEN, _BATCH), jnp.bfloat16),
    )(g_tm, W1p, b1)


def _out_body(ht_ref, w2t_ref, b2_ref, out_ref):
    out_ref[...] = (
        jnp.dot(w2t_ref[...].astype(jnp.bfloat16), ht_ref[...],
                preferred_element_type=jnp.float32)
        + b2_ref[...][:, None])


def _out_layer_t(ht, W2t, b2):
    nblk = pl.cdiv(_VOCAB, _BN)
    return pl.pallas_call(
        _out_body,
        grid=(nblk,),
        in_specs=[
            pl.BlockSpec((_HIDDEN, _BATCH), lambda j: (0, 0)),
            pl.BlockSpec((_BN, _HIDDEN), lambda j: (j, 0)),
            pl.BlockSpec((_BN,), lambda j: (j,)),
        ],
        out_specs=pl.BlockSpec((_BN, _BATCH), lambda j: (j, 0)),
        out_shape=jax.ShapeDtypeStruct((_VOCAB, _BATCH), jnp.float32),
        compiler_params=pltpu.CompilerParams(
            dimension_semantics=("arbitrary",)),
    )(ht, W2t, b2)


def kernel(inputs, embed_table, W1, b1, W2, b2):
    flat_idx_tm = inputs.T.reshape(-1)
    table_pad = _padt(embed_table.T)
    g_tm = _sc_gather(table_pad, flat_idx_tm)
    W1p = jnp.pad(W1.reshape(_NPREV, _EMBED, _HIDDEN),
                  ((0, 0), (0, _ROW - _EMBED), (0, 0))).astype(jnp.bfloat16)
    ht = _h_layer(g_tm, W1p, b1)
    out_t = _out_layer_t(ht, W2.T, b2)
    return out_t.T


# speedup vs baseline: 3.1733x; 1.0066x over previous
import jax
import jax.numpy as jnp
from jax.experimental import pallas as pl
from jax.experimental.pallas import tpu as pltpu
from jax.experimental.pallas import tpu_sc as plsc

_VOCAB = 100000
_EMBED = 64
_HIDDEN = 512
_NPREV = 20
_BATCH = 1024

_GW = 128
_BN = 3072
_ROW = 2 * _EMBED


def _sc_gather(table_pad, flat_idx):
    n = flat_idx.shape[0]
    idx2 = flat_idx.reshape(1, n)
    mesh = plsc.VectorSubcoreMesh(core_axis_name="c", subcore_axis_name="s")

    @pl.kernel(out_type=jax.ShapeDtypeStruct((n, _ROW), table_pad.dtype),
               mesh=mesh)
    def gather_kernel(table_hbm, idx_hbm, out_hbm):
        def body(i_vmem, o_vmem):
            pltpu.sync_copy(table_hbm.at[i_vmem.at[0]], o_vmem)

        pltpu.emit_pipeline(
            body,
            grid=(n // _GW,),
            in_specs=[pl.BlockSpec((1, _GW), lambda i: (0, i))],
            out_specs=[pl.BlockSpec((_GW, _ROW), lambda i: (i, 0))],
            core_axis_name=("c", "s"),
            dimension_semantics=(pltpu.PARALLEL,),
        )(idx_hbm, out_hbm)

    return gather_kernel(table_pad, idx2)


_CB = 16384


def _padt_body(tt_ref, out_ref):
    out_ref[:, 0:_EMBED] = tt_ref[...].T
    out_ref[:, _EMBED:] = jnp.zeros((_CB, _ROW - _EMBED), jnp.float32)


def _padt(tT):
    return pl.pallas_call(
        _padt_body,
        grid=(pl.cdiv(_VOCAB, _CB),),
        in_specs=[pl.BlockSpec((_EMBED, _CB), lambda j: (0, j))],
        out_specs=pl.BlockSpec((_CB, _ROW), lambda j: (j, 0)),
        out_shape=jax.ShapeDtypeStruct((_VOCAB, _ROW), jnp.float32),
        compiler_params=pltpu.CompilerParams(
            dimension_semantics=("arbitrary",)),
    )(tT)


def _h_body(g_ref, w1p_ref, b1_ref, ht_ref):
    acc = jnp.zeros((_BATCH, _HIDDEN), jnp.float32)
    for t in range(_NPREV):
        gt = g_ref[t * _BATCH:(t + 1) * _BATCH, :]
        acc = acc + jnp.dot(gt.astype(jnp.bfloat16), w1p_ref[t],
                            preferred_element_type=jnp.float32)
    h = jnp.tanh(acc + b1_ref[...])
    ht_ref[...] = h.T.astype(jnp.bfloat16)


def _h_layer(g_tm, W1p, b1):
    return pl.pallas_call(
        _h_body,
        out_shape=jax.ShapeDtypeStruct((_HIDDEN, _BATCH), jnp.bfloat16),
    )(g_tm, W1p, b1)


def _out_body(ht_ref, w2t_ref, b2_ref, out_ref):
    out_ref[...] = (
        jnp.dot(w2t_ref[...].astype(jnp.bfloat16), ht_ref[...],
                preferred_element_type=jnp.float32)
        + b2_ref[...][:, None])


def _out_layer_t(ht, W2t, b2):
    nblk = pl.cdiv(_VOCAB, _BN)
    return pl.pallas_call(
        _out_body,
        grid=(nblk,),
        in_specs=[
            pl.BlockSpec((_HIDDEN, _BATCH), lambda j: (0, 0)),
            pl.BlockSpec((_BN, _HIDDEN), lambda j: (j, 0)),
            pl.BlockSpec((_BN,), lambda j: (j,)),
        ],
        out_specs=pl.BlockSpec((_BN, _BATCH), lambda j: (j, 0)),
        out_shape=jax.ShapeDtypeStruct((_VOCAB, _BATCH), jnp.float32),
        compiler_params=pltpu.CompilerParams(
            dimension_semantics=("arbitrary",)),
    )(ht, W2t, b2)


def kernel(inputs, embed_table, W1, b1, W2, b2):
    flat_idx_tm = inputs.T.reshape(-1)
    table_pad = _padt(embed_table.T)
    g_tm = _sc_gather(table_pad, flat_idx_tm)
    W1p = jnp.pad(W1.reshape(_NPREV, _EMBED, _HIDDEN),
                  ((0, 0), (0, _ROW - _EMBED), (0, 0))).astype(jnp.bfloat16)
    ht = _h_layer(g_tm, W1p, b1)
    out_t = _out_layer_t(ht, W2.T, b2)
    return out_t.T
